# baseline (speedup 1.0000x reference)
import functools

import jax
import jax.numpy as jnp
from jax import lax
from jax.experimental import pallas as pl
from jax.experimental.pallas import tpu as pltpu
from jax.experimental.pallas import tpu_sc as plsc


@functools.cache
def _build(V, D, BH, C):
    info = plsc.get_sparse_core_info()
    NC, NS, L = info.num_cores, info.num_subcores, info.num_lanes
    NW = NC * NS
    b_per_w = BH // NW
    n_chunks = b_per_w // C
    assert BH % (NW * C) == 0 and C % L == 0

    mesh = plsc.VectorSubcoreMesh(core_axis_name="c", subcore_axis_name="s")

    @functools.partial(
        pl.kernel,
        mesh=mesh,
        compiler_params=pltpu.CompilerParams(use_tc_tiling_on_sc=False),
        out_type=jax.ShapeDtypeStruct((BH, D), jnp.float32),
        scratch_types=[
            pltpu.VMEM((C,), jnp.int32),
            pltpu.VMEM((C, D), jnp.float32),
            pltpu.VMEM((L,), jnp.int32),
            pltpu.SemaphoreType.DMA,
        ],
    )
    def k(ids_hbm, off_hbm, table_hbm, out_hbm, idx_v, rows_v, off_v, sem):
        wid = lax.axis_index("s") * NC + lax.axis_index("c")
        base = wid * b_per_w
        pltpu.sync_copy(off_hbm, off_v)
        off_vec = off_v[...]

        def chunk(c, _):
            start = base + c * C

            pltpu.sync_copy(ids_hbm.at[pl.ds(start, C)], idx_v)

            def add_off(j, _):
                sl = pl.ds(pl.multiple_of(j * L, L), L)
                idx_v[sl] = idx_v[sl] + off_vec
                return 0

            lax.fori_loop(0, C // L, add_off, 0)

            pltpu.async_copy(table_hbm.at[idx_v], rows_v, sem).wait()
            pltpu.sync_copy(rows_v, out_hbm.at[pl.ds(start, C)])
            return 0

        lax.fori_loop(0, n_chunks, chunk, 0)

    return k


def kernel(input_ids, offsets, table):
    B, H = input_ids.shape
    V, D = table.shape
    BH = B * H
    L = 16
    ids_flat = input_ids.reshape(BH)
    off16 = jnp.tile(offsets, L // H)
    out = _build(V, D, BH, 1024)(ids_flat, off16, table)
    return out.reshape(B, H, D)
